# baseline (speedup 1.0000x reference)
import functools

import numpy as np
import jax
import jax.numpy as jnp
from jax import lax
from jax.experimental import pallas as pl
from jax.experimental.pallas import tpu as pltpu
from jax.experimental.pallas import tpu_sc as plsc

C = 192
HW = 224 * 224
P = 8192
NPAIR = 2 * P
NW = 32
PAIRS_PER_W = NPAIR // NW
CHUNK = 128
NCHUNK = PAIRS_PER_W // CHUNK
CB = C // 16

_rng = np.random.RandomState(0)
_RC1 = _rng.choice(HW, P).astype(np.int32)
_RC2 = _rng.choice(HW, P).astype(np.int32)

H = 224
W = 224
HB = 16
_TN = H // HB
CPAD = 256


NWORD = C // 2
WPAD = 128


def _tr_body(x1_ref, x2_ref, o1_ref, o2_ref):
    for x_ref, o_ref in ((x1_ref, o1_ref), (x2_ref, o2_ref)):
        x = x_ref[...]
        lo = x[:NWORD].astype(jnp.bfloat16)
        hi = x[NWORD:].astype(jnp.bfloat16)
        lo_u = lax.convert_element_type(
            lax.bitcast_convert_type(lo, jnp.uint16), jnp.uint32)
        hi_u = lax.convert_element_type(
            lax.bitcast_convert_type(hi, jnp.uint16), jnp.uint32)
        wf = lax.bitcast_convert_type(lo_u | (hi_u << 16), jnp.float32)
        for h in range(HB):
            o_ref[pl.ds(h * W, W), pl.ds(0, NWORD)] = wf[:, h, :].T


_transpose2 = pl.pallas_call(
    _tr_body,
    grid=(_TN,),
    in_specs=[
        pl.BlockSpec((C, HB, W), lambda i: (0, i, 0)),
        pl.BlockSpec((C, HB, W), lambda i: (0, i, 0)),
    ],
    out_specs=[
        pl.BlockSpec((HB * W, WPAD), lambda i: (i, 0)),
        pl.BlockSpec((HB * W, WPAD), lambda i: (i, 0)),
    ],
    out_shape=[jax.ShapeDtypeStruct((HW, WPAD), jnp.float32)] * 2,
)

_mesh = plsc.VectorSubcoreMesh(core_axis_name="c", subcore_axis_name="s")
GCHUNK = 128
NGCHUNK = PAIRS_PER_W // GCHUNK


def _rsqrt16(x):
    i = plsc.bitcast(x, jnp.int32)
    y = plsc.bitcast(jnp.int32(0x5F3759DF) - (i >> 1), jnp.float32)
    for _ in range(3):
        y = y * (1.5 - 0.5 * x * y * y)
    return y


@functools.partial(
    pl.kernel,
    out_type=jax.ShapeDtypeStruct((NW, 1, 16), jnp.float32),
    mesh=_mesh,
    scratch_types=[
        pltpu.VMEM((NCHUNK, CHUNK), jnp.int32),
        pltpu.VMEM((NCHUNK, CHUNK), jnp.int32),
        pltpu.VMEM((3, GCHUNK, WPAD), jnp.float32),
        pltpu.VMEM((3, GCHUNK, WPAD), jnp.float32),
        pltpu.VMEM((1, 16), jnp.float32),
        pltpu.SemaphoreType.DMA,
        pltpu.SemaphoreType.DMA,
        pltpu.SemaphoreType.DMA,
    ],
    compiler_params=pltpu.CompilerParams(
        use_tc_tiling_on_sc=True, needs_layout_passes=False),
)
def _sc_ssq(f1t, f2t, idx1, idx2, out,
            idx1_v, idx2_v, rows1_v, rows2_v, sq_v, semA, semB, semC):
    wid = lax.axis_index("s") * 2 + lax.axis_index("c")
    pltpu.sync_copy(idx1.at[wid], idx1_v)
    pltpu.sync_copy(idx2.at[wid], idx2_v)
    sems = [semA, semB, semC]
    RING = 3
    iota16 = lax.iota(jnp.int32, 16)

    def start(c):
        slot = c % RING
        cp1 = pltpu.async_copy(f1t.at[idx1_v.at[c]], rows1_v.at[slot], sems[slot])
        cp2 = pltpu.async_copy(f2t.at[idx2_v.at[c]], rows2_v.at[slot], sems[slot])
        return cp1, cp2

    def compute(slot):
        r1 = rows1_v.at[slot]
        r2 = rows2_v.at[slot]

        def group(g, sqsum):
            pend = jnp.zeros((16,), jnp.float32)
            for pp in range(16):
                p = g * 16 + pp
                acc = jnp.zeros((16,), jnp.float32)
                for cb in range(NWORD // 16):
                    w1 = plsc.bitcast(r1[p, pl.ds(cb * 16, 16)], jnp.int32)
                    w2 = plsc.bitcast(r2[p, pl.ds(cb * 16, 16)], jnp.int32)
                    dlo = (plsc.bitcast(w1 << 16, jnp.float32)
                           - plsc.bitcast(w2 << 16, jnp.float32))
                    dhi = (plsc.bitcast(w1 & jnp.int32(-65536), jnp.float32)
                           - plsc.bitcast(w2 & jnp.int32(-65536), jnp.float32))
                    acc = acc + dlo * dlo + dhi * dhi
                pend = jnp.where(iota16 == pp, jnp.sum(acc), pend)
            x = pend + 1e-7
            return sqsum + x * _rsqrt16(x)

        return lax.fori_loop(0, GCHUNK // 16, group,
                             jnp.zeros((16,), jnp.float32))

    cps = [None] * RING
    started = 0
    total = None
    for c in range(NGCHUNK):
        while started < min(c + RING, NGCHUNK):
            cps[started % RING] = start(started)
            started += 1
        cp1, cp2 = cps[c % RING]
        cp1.wait()
        cp2.wait()
        s = compute(c % RING)
        total = s if total is None else total + s
    sq_v[0, :] = total
    pltpu.sync_copy(sq_v, out.at[wid])


def _fin_body(sq_ref, o_ref):
    x = sq_ref[...]
    mp = jnp.sum(x[: NW // 2]) / P
    mn = jnp.sum(x[NW // 2 :]) / P
    loss = mp + jnp.maximum(1.0 - mn, 0.0)
    o_ref[...] = jnp.reshape(loss, (1, 1))


_final = pl.pallas_call(
    _fin_body,
    out_shape=jax.ShapeDtypeStruct((1, 1), jnp.float32),
)


def kernel(feats1, feats2, pos_pairs):
    f1t, f2t = _transpose2(feats1, feats2)
    idx1 = jnp.concatenate(
        [pos_pairs[0, :, 0], jnp.asarray(_RC1)]).reshape(NW, NCHUNK, CHUNK)
    idx2 = jnp.concatenate(
        [pos_pairs[0, :, 1], jnp.asarray(_RC2)]).reshape(NW, NCHUNK, CHUNK)
    ssq = _sc_ssq(f1t, f2t, idx1, idx2)
    return _final(ssq)[0, 0]


# speedup vs baseline: 1.0098x; 1.0098x over previous
import functools

import numpy as np
import jax
import jax.numpy as jnp
from jax import lax
from jax.experimental import pallas as pl
from jax.experimental.pallas import tpu as pltpu
from jax.experimental.pallas import tpu_sc as plsc

C = 192
HW = 224 * 224
P = 8192
NPAIR = 2 * P
NW = 32
PAIRS_PER_W = NPAIR // NW
CHUNK = 128
NCHUNK = PAIRS_PER_W // CHUNK
CB = C // 16

_rng = np.random.RandomState(0)
_RC1 = _rng.choice(HW, P).astype(np.int32)
_RC2 = _rng.choice(HW, P).astype(np.int32)

H = 224
W = 224
HB = 32
_TN = H // HB
CPAD = 256


NWORD = C // 2
WPAD = 128


def _tr_body(x1_ref, x2_ref, o1_ref, o2_ref):
    for x_ref, o_ref in ((x1_ref, o1_ref), (x2_ref, o2_ref)):
        x = x_ref[...]
        lo = x[:NWORD].astype(jnp.bfloat16)
        hi = x[NWORD:].astype(jnp.bfloat16)
        lo_u = lax.convert_element_type(
            lax.bitcast_convert_type(lo, jnp.uint16), jnp.uint32)
        hi_u = lax.convert_element_type(
            lax.bitcast_convert_type(hi, jnp.uint16), jnp.uint32)
        wf = lax.bitcast_convert_type(lo_u | (hi_u << 16), jnp.float32)
        for h in range(HB):
            o_ref[pl.ds(h * W, W), pl.ds(0, NWORD)] = wf[:, h, :].T


_transpose2 = pl.pallas_call(
    _tr_body,
    grid=(_TN,),
    in_specs=[
        pl.BlockSpec((C, HB, W), lambda i: (0, i, 0)),
        pl.BlockSpec((C, HB, W), lambda i: (0, i, 0)),
    ],
    out_specs=[
        pl.BlockSpec((HB * W, WPAD), lambda i: (i, 0)),
        pl.BlockSpec((HB * W, WPAD), lambda i: (i, 0)),
    ],
    out_shape=[jax.ShapeDtypeStruct((HW, WPAD), jnp.float32)] * 2,
)

_mesh = plsc.VectorSubcoreMesh(core_axis_name="c", subcore_axis_name="s")
GCHUNK = 128
NGCHUNK = PAIRS_PER_W // GCHUNK


def _rsqrt16(x):
    i = plsc.bitcast(x, jnp.int32)
    y = plsc.bitcast(jnp.int32(0x5F3759DF) - (i >> 1), jnp.float32)
    for _ in range(3):
        y = y * (1.5 - 0.5 * x * y * y)
    return y


@functools.partial(
    pl.kernel,
    out_type=jax.ShapeDtypeStruct((NW, 1, 16), jnp.float32),
    mesh=_mesh,
    scratch_types=[
        pltpu.VMEM((NCHUNK, CHUNK), jnp.int32),
        pltpu.VMEM((NCHUNK, CHUNK), jnp.int32),
        pltpu.VMEM((2, GCHUNK, WPAD), jnp.float32),
        pltpu.VMEM((2, GCHUNK, WPAD), jnp.float32),
        pltpu.VMEM((1, 16), jnp.float32),
        pltpu.SemaphoreType.DMA,
        pltpu.SemaphoreType.DMA,
    ],
    compiler_params=pltpu.CompilerParams(
        use_tc_tiling_on_sc=True, needs_layout_passes=False),
)
def _sc_ssq(f1t, f2t, idx1, idx2, out,
            idx1_v, idx2_v, rows1_v, rows2_v, sq_v, semA, semB):
    wid = lax.axis_index("s") * 2 + lax.axis_index("c")
    pltpu.sync_copy(idx1.at[wid], idx1_v)
    pltpu.sync_copy(idx2.at[wid], idx2_v)
    sems = [semA, semB]
    RING = 2
    iota16 = lax.iota(jnp.int32, 16)

    def start(c):
        slot = c % RING
        cp1 = pltpu.async_copy(f1t.at[idx1_v.at[c]], rows1_v.at[slot], sems[slot])
        cp2 = pltpu.async_copy(f2t.at[idx2_v.at[c]], rows2_v.at[slot], sems[slot])
        return cp1, cp2

    def compute(slot):
        r1 = rows1_v.at[slot]
        r2 = rows2_v.at[slot]

        def group(g, sqsum):
            pend = jnp.zeros((16,), jnp.float32)
            for pp in range(16):
                p = g * 16 + pp
                acc = jnp.zeros((16,), jnp.float32)
                for cb in range(NWORD // 16):
                    w1 = plsc.bitcast(r1[p, pl.ds(cb * 16, 16)], jnp.int32)
                    w2 = plsc.bitcast(r2[p, pl.ds(cb * 16, 16)], jnp.int32)
                    dlo = (plsc.bitcast(w1 << 16, jnp.float32)
                           - plsc.bitcast(w2 << 16, jnp.float32))
                    dhi = (plsc.bitcast(w1 & jnp.int32(-65536), jnp.float32)
                           - plsc.bitcast(w2 & jnp.int32(-65536), jnp.float32))
                    acc = acc + dlo * dlo + dhi * dhi
                pend = jnp.where(iota16 == pp, jnp.sum(acc), pend)
            x = pend + 1e-7
            return sqsum + x * _rsqrt16(x)

        return lax.fori_loop(0, GCHUNK // 16, group,
                             jnp.zeros((16,), jnp.float32))

    cps = [None] * RING
    started = 0
    total = None
    for c in range(NGCHUNK):
        while started < min(c + RING, NGCHUNK):
            cps[started % RING] = start(started)
            started += 1
        cp1, cp2 = cps[c % RING]
        cp1.wait()
        cp2.wait()
        s = compute(c % RING)
        total = s if total is None else total + s
    sq_v[0, :] = total
    pltpu.sync_copy(sq_v, out.at[wid])


def _fin_body(sq_ref, o_ref):
    x = sq_ref[...]
    mp = jnp.sum(x[: NW // 2]) / P
    mn = jnp.sum(x[NW // 2 :]) / P
    loss = mp + jnp.maximum(1.0 - mn, 0.0)
    o_ref[...] = jnp.reshape(loss, (1, 1))


_final = pl.pallas_call(
    _fin_body,
    out_shape=jax.ShapeDtypeStruct((1, 1), jnp.float32),
)


def kernel(feats1, feats2, pos_pairs):
    f1t, f2t = _transpose2(feats1, feats2)
    idx1 = jnp.concatenate(
        [pos_pairs[0, :, 0], jnp.asarray(_RC1)]).reshape(NW, NCHUNK, CHUNK)
    idx2 = jnp.concatenate(
        [pos_pairs[0, :, 1], jnp.asarray(_RC2)]).reshape(NW, NCHUNK, CHUNK)
    ssq = _sc_ssq(f1t, f2t, idx1, idx2)
    return _final(ssq)[0, 0]


# speedup vs baseline: 1.0161x; 1.0062x over previous
import functools

import numpy as np
import jax
import jax.numpy as jnp
from jax import lax
from jax.experimental import pallas as pl
from jax.experimental.pallas import tpu as pltpu
from jax.experimental.pallas import tpu_sc as plsc

C = 192
HW = 224 * 224
P = 8192
NPAIR = 2 * P
NW = 32
PAIRS_PER_W = NPAIR // NW
CHUNK = 128
NCHUNK = PAIRS_PER_W // CHUNK
CB = C // 16

_rng = np.random.RandomState(0)
_RC1 = _rng.choice(HW, P).astype(np.int32)
_RC2 = _rng.choice(HW, P).astype(np.int32)

H = 224
W = 224
HB = 16
_TN = H // HB
NWORD = C // 2
WPAD = 128


def _tr_body(x1_ref, x2_ref, o1_ref, o2_ref):
    for x_ref, o_ref in ((x1_ref, o1_ref), (x2_ref, o2_ref)):
        x = x_ref[...]
        lo = x[:NWORD].astype(jnp.bfloat16)
        hi = x[NWORD:].astype(jnp.bfloat16)
        lo_u = lax.convert_element_type(
            lax.bitcast_convert_type(lo, jnp.uint16), jnp.uint32)
        hi_u = lax.convert_element_type(
            lax.bitcast_convert_type(hi, jnp.uint16), jnp.uint32)
        wf = lax.bitcast_convert_type(lo_u | (hi_u << 16), jnp.float32)
        for h in range(HB):
            o_ref[pl.ds(h * W, W), pl.ds(0, NWORD)] = wf[:, h, :].T


_transpose2 = pl.pallas_call(
    _tr_body,
    grid=(_TN,),
    in_specs=[
        pl.BlockSpec((C, HB, W), lambda i: (0, i, 0)),
        pl.BlockSpec((C, HB, W), lambda i: (0, i, 0)),
    ],
    out_specs=[
        pl.BlockSpec((HB * W, WPAD), lambda i: (i, 0)),
        pl.BlockSpec((HB * W, WPAD), lambda i: (i, 0)),
    ],
    out_shape=[jax.ShapeDtypeStruct((HW, WPAD), jnp.float32)] * 2,
)

_mesh = plsc.VectorSubcoreMesh(core_axis_name="c", subcore_axis_name="s")
GCHUNK = 128
NGCHUNK = PAIRS_PER_W // GCHUNK


def _rsqrt16(x):
    i = plsc.bitcast(x, jnp.int32)
    y = plsc.bitcast(jnp.int32(0x5F3759DF) - (i >> 1), jnp.float32)
    for _ in range(3):
        y = y * (1.5 - 0.5 * x * y * y)
    return y


@functools.partial(
    pl.kernel,
    out_type=jax.ShapeDtypeStruct((NW, 1, 16), jnp.float32),
    mesh=_mesh,
    scratch_types=[
        pltpu.VMEM((NCHUNK, CHUNK), jnp.int32),
        pltpu.VMEM((NCHUNK, CHUNK), jnp.int32),
        pltpu.VMEM((2, GCHUNK, WPAD), jnp.float32),
        pltpu.VMEM((2, GCHUNK, WPAD), jnp.float32),
        pltpu.VMEM((1, 16), jnp.float32),
        pltpu.SemaphoreType.DMA,
        pltpu.SemaphoreType.DMA,
    ],
    compiler_params=pltpu.CompilerParams(
        use_tc_tiling_on_sc=True, needs_layout_passes=False),
)
def _sc_ssq(f1t, f2t, idx1, idx2, out,
            idx1_v, idx2_v, rows1_v, rows2_v, sq_v, semA, semB):
    wid = lax.axis_index("s") * 2 + lax.axis_index("c")
    pltpu.sync_copy(idx1.at[wid], idx1_v)
    pltpu.sync_copy(idx2.at[wid], idx2_v)
    sems = [semA, semB]
    RING = 2
    iota16 = lax.iota(jnp.int32, 16)

    def start(c):
        slot = c % RING
        cp1 = pltpu.async_copy(f1t.at[idx1_v.at[c]], rows1_v.at[slot], sems[slot])
        cp2 = pltpu.async_copy(f2t.at[idx2_v.at[c]], rows2_v.at[slot], sems[slot])
        return cp1, cp2

    def compute(slot):
        r1 = rows1_v.at[slot]
        r2 = rows2_v.at[slot]

        def group(g, sqsum):
            pend = jnp.zeros((16,), jnp.float32)
            for pp in range(16):
                p = g * 16 + pp
                acc = jnp.zeros((16,), jnp.float32)
                for cb in range(NWORD // 16):
                    w1 = plsc.bitcast(r1[p, pl.ds(cb * 16, 16)], jnp.int32)
                    w2 = plsc.bitcast(r2[p, pl.ds(cb * 16, 16)], jnp.int32)
                    dlo = (plsc.bitcast(w1 << 16, jnp.float32)
                           - plsc.bitcast(w2 << 16, jnp.float32))
                    dhi = (plsc.bitcast(w1 & jnp.int32(-65536), jnp.float32)
                           - plsc.bitcast(w2 & jnp.int32(-65536), jnp.float32))
                    acc = acc + dlo * dlo + dhi * dhi
                pend = jnp.where(iota16 == pp, jnp.sum(acc), pend)
            x = pend + 1e-7
            return sqsum + x * _rsqrt16(x)

        return lax.fori_loop(0, GCHUNK // 16, group,
                             jnp.zeros((16,), jnp.float32))

    cps = [None] * RING
    started = 0
    total = None
    for c in range(NGCHUNK):
        while started < min(c + RING, NGCHUNK):
            cps[started % RING] = start(started)
            started += 1
        cp1, cp2 = cps[c % RING]
        cp1.wait()
        cp2.wait()
        s = compute(c % RING)
        total = s if total is None else total + s
    sq_v[0, :] = total
    pltpu.sync_copy(sq_v, out.at[wid])


def _fin_body(sq_ref, o_ref):
    x = sq_ref[...]
    mp = jnp.sum(x[: NW // 2]) / P
    mn = jnp.sum(x[NW // 2 :]) / P
    loss = mp + jnp.maximum(1.0 - mn, 0.0)
    o_ref[...] = jnp.reshape(loss, (1, 1))


_final = pl.pallas_call(
    _fin_body,
    out_shape=jax.ShapeDtypeStruct((1, 1), jnp.float32),
)


def kernel(feats1, feats2, pos_pairs):
    f1t, f2t = _transpose2(feats1, feats2)
    idx1 = jnp.concatenate(
        [pos_pairs[0, :, 0], jnp.asarray(_RC1)]).reshape(NW, NCHUNK, CHUNK)
    idx2 = jnp.concatenate(
        [pos_pairs[0, :, 1], jnp.asarray(_RC2)]).reshape(NW, NCHUNK, CHUNK)
    ssq = _sc_ssq(f1t, f2t, idx1, idx2)
    return _final(ssq)[0, 0]
